# baseline (speedup 1.0000x reference)
import functools
import math

import jax
import jax.numpy as jnp
import numpy as np
from jax import lax
from jax.experimental import pallas as pl
from jax.experimental.pallas import tpu as pltpu
from jax.experimental.pallas import tpu_sc as plsc

D = 64
V = 1000000
SCALE = np.float32(math.sqrt(D))

_NC = 2
_NS = 16
_NW = _NC * _NS

_FULL_COLS = V // 128
_TAIL_ROWS = V - _FULL_COLS * 128
_NPAIR = V // 2


def _make_k1():
    mesh = plsc.VectorSubcoreMesh(core_axis_name="c", subcore_axis_name="s")

    @functools.partial(
        pl.kernel,
        mesh=mesh,
        compiler_params=pltpu.CompilerParams(needs_layout_passes=False),
        out_type=jax.ShapeDtypeStruct((_NPAIR, 128), jnp.float32),
        scratch_types=[
            pltpu.VMEM((64, 128), jnp.float32),
            pltpu.VMEM((64, 128), jnp.float32),
            pltpu.VMEM((32, 128), jnp.float32),
        ],
    )
    def k1(tt_hbm, tail_hbm, lin_hbm, vbuf, nbuf, tbuf):
        wid = lax.axis_index("s") * _NC + lax.axis_index("c")

        @pl.when(wid == 0)
        def _():
            pltpu.sync_copy(tail_hbm, tbuf)
            pltpu.sync_copy(tbuf, lin_hbm.at[pl.ds(_FULL_COLS * 64, 32)])

        iota = lax.iota(jnp.int32, 16)
        n_cols = jnp.where(wid < _FULL_COLS % _NW,
                           _FULL_COLS // _NW + 1, _FULL_COLS // _NW)

        def col_body(m, _):
            tc = wid + m * _NW
            pltpu.sync_copy(tt_hbm.at[:, pl.ds(tc * 128, 128)], vbuf)

            def pair_body(q, _):
                for k in range(8):
                    ridx = iota + (16 * (k % 4))
                    cidx = jnp.full((16,), 2 * q + (1 if k >= 4 else 0),
                                    jnp.int32)
                    vals = plsc.load_gather(vbuf, [ridx, cidx]) * SCALE
                    nbuf[q, pl.ds(16 * k, 16)] = vals
                return 0

            lax.fori_loop(0, 64, pair_body, 0)
            pltpu.sync_copy(nbuf, lin_hbm.at[pl.ds(tc * 64, 64)])
            return 0

        lax.fori_loop(0, n_cols, col_body, 0)

    return k1


def _make_k2(NJ: int, NI: int):
    njb = NJ // 8
    nti = NI // 128
    units = njb * nti
    per_w = units // _NW
    mesh = plsc.VectorSubcoreMesh(core_axis_name="c", subcore_axis_name="s")

    @functools.partial(
        pl.kernel,
        mesh=mesh,
        compiler_params=pltpu.CompilerParams(needs_layout_passes=False),
        out_type=jax.ShapeDtypeStruct((NJ, D, NI), jnp.float32),
        scratch_types=[
            pltpu.VMEM((8, 128), jnp.int32),
            pltpu.VMEM((128,), jnp.int32),
            pltpu.VMEM((128,), jnp.int32),
            pltpu.VMEM((128, 128), jnp.float32),
            pltpu.VMEM((64, 128), jnp.float32),
            pltpu.SemaphoreType.DMA,
        ],
    )
    def k2(xt_hbm, lin_hbm, out_hbm, idxt, pidx, par64, pairbuf, obuf, sem):
        wid = lax.axis_index("s") * _NC + lax.axis_index("c")
        iota = lax.iota(jnp.int32, 16)

        def unit_body(u, _):
            t = wid * per_w + u
            jb = t // nti
            tc = t % nti
            pltpu.sync_copy(
                xt_hbm.at[pl.ds(jb * 8, 8), pl.ds(tc * 128, 128)], idxt)

            def j_body(j8, _):
                def prep(m, _):
                    v = idxt[j8, pl.ds(16 * m, 16)]
                    pidx[pl.ds(16 * m, 16)] = lax.shift_right_logical(v, 1)
                    par64[pl.ds(16 * m, 16)] = (v & 1) * 64
                    return 0

                lax.fori_loop(0, 8, prep, 0)
                pltpu.async_copy(lin_hbm.at[pidx], pairbuf, sem).wait()

                def grp_body(m, _):
                    parv = par64[pl.ds(16 * m, 16)]
                    ridx = iota + 16 * m

                    def d_body(d, _):
                        obuf[d, pl.ds(16 * m, 16)] = plsc.load_gather(
                            pairbuf, [ridx, parv + d])
                        return 0

                    lax.fori_loop(0, D, d_body, 0)
                    return 0

                lax.fori_loop(0, 8, grp_body, 0)
                pltpu.sync_copy(
                    obuf, out_hbm.at[jb * 8 + j8, :, pl.ds(tc * 128, 128)])
                return 0

            lax.fori_loop(0, 8, j_body, 0)
            return 0

        lax.fori_loop(0, per_w, unit_body, 0)

    return k2


def kernel(x, table):
    NI, NJ = x.shape
    xt = jnp.transpose(x.astype(jnp.int32))
    tt = jnp.transpose(table)
    tail = jnp.reshape(table[_FULL_COLS * 128:] * SCALE, (32, 128))
    lin = _make_k1()(tt, tail)
    outt = _make_k2(NJ, NI)(xt, lin)
    return jnp.transpose(outt, (2, 0, 1))


# speedup vs baseline: 1.2149x; 1.2149x over previous
import functools
import math

import jax
import jax.numpy as jnp
import numpy as np
from jax import lax
from jax.experimental import pallas as pl
from jax.experimental.pallas import tpu as pltpu
from jax.experimental.pallas import tpu_sc as plsc

D = 64
V = 1000000
SCALE = np.float32(math.sqrt(D))

_NC = 2
_NS = 16
_NW = _NC * _NS

_FULL_COLS = V // 128
_NPAIR = V // 2
_G = 4
_NBLK = _FULL_COLS // _G


def _make_k1():
    mesh = plsc.VectorSubcoreMesh(core_axis_name="c", subcore_axis_name="s")

    @functools.partial(
        pl.kernel,
        mesh=mesh,
        compiler_params=pltpu.CompilerParams(needs_layout_passes=False),
        out_type=jax.ShapeDtypeStruct((_NPAIR, 128), jnp.float32),
        scratch_types=[
            pltpu.VMEM((64, 128 * _G), jnp.float32),
            pltpu.VMEM((64, 128 * _G), jnp.float32),
            pltpu.VMEM((64 * _G, 128), jnp.float32),
            pltpu.VMEM((32, 128), jnp.float32),
            pltpu.SemaphoreType.DMA,
            pltpu.SemaphoreType.DMA,
            pltpu.SemaphoreType.DMA,
        ],
    )
    def k1(tt_hbm, tail_hbm, lin_hbm, vb0, vb1, nb, tbuf, gi0, gi1, go):
        wid = lax.axis_index("s") * _NC + lax.axis_index("c")

        @pl.when(wid == 0)
        def _():
            pltpu.sync_copy(tail_hbm, tbuf)
            pltpu.sync_copy(tbuf, lin_hbm.at[pl.ds(_FULL_COLS * 64, 32)])

        iota = lax.iota(jnp.int32, 16)
        n_b = jnp.where(wid < _NBLK % _NW, _NBLK // _NW + 1, _NBLK // _NW)

        def fire_in(m, vb, sem):
            blk = wid + m * _NW
            pltpu.async_copy(tt_hbm.at[:, pl.ds(blk * 128 * _G, 128 * _G)],
                             vb, sem)

        def transpose(vb):
            def pair_body(t, _):
                for u in range(2):
                    qq = 2 * t + u
                    c0 = jnp.full((16,), 2 * qq, jnp.int32)
                    c1 = c0 + 1
                    for k in range(8):
                        ridx = iota + (16 * (k % 4))
                        cidx = c1 if k >= 4 else c0
                        nb[qq, pl.ds(16 * k, 16)] = (
                            plsc.load_gather(vb, [ridx, cidx]) * SCALE)
                return 0

            lax.fori_loop(0, 32 * _G, pair_body, 0)

        def half(m, vb, gi):
            @pl.when(m < n_b)
            def _():
                blk = wid + m * _NW
                pltpu.make_async_copy(
                    tt_hbm.at[:, pl.ds(blk * 128 * _G, 128 * _G)], vb,
                    gi).wait()

                @pl.when(m >= 1)
                def _():
                    pltpu.make_async_copy(
                        nb, lin_hbm.at[pl.ds(blk * 64 * _G, 64 * _G)],
                        go).wait()

                transpose(vb)
                pltpu.async_copy(
                    nb, lin_hbm.at[pl.ds(blk * 64 * _G, 64 * _G)], go)

            @pl.when(m + 2 < n_b)
            def _():
                fire_in(m + 2, vb, gi)

        fire_in(0, vb0, gi0)
        fire_in(1, vb1, gi1)

        def body(h, _):
            half(2 * h, vb0, gi0)
            half(2 * h + 1, vb1, gi1)
            return 0

        lax.fori_loop(0, (_NBLK // _NW + 2) // 2, body, 0)
        pltpu.make_async_copy(nb, lin_hbm.at[pl.ds(0, 64 * _G)], go).wait()

    return k1


def _make_k2(NJ: int, NI: int):
    njb = NJ // 8
    nti = NI // 128
    units = njb * nti
    per_w = units // _NW
    n_t = per_w * 8
    mesh = plsc.VectorSubcoreMesh(core_axis_name="c", subcore_axis_name="s")

    @functools.partial(
        pl.kernel,
        mesh=mesh,
        compiler_params=pltpu.CompilerParams(needs_layout_passes=False),
        out_type=jax.ShapeDtypeStruct((NJ, D, NI), jnp.float32),
        scratch_types=[
            pltpu.VMEM((8, per_w * 128), jnp.int32),
            pltpu.VMEM((3, 128), jnp.int32),
            pltpu.VMEM((3, 128), jnp.int32),
            pltpu.VMEM((128, 128), jnp.float32),
            pltpu.VMEM((128, 128), jnp.float32),
            pltpu.VMEM((128, 128), jnp.float32),
            pltpu.VMEM((64, 128), jnp.float32),
            pltpu.VMEM((64, 128), jnp.float32),
            pltpu.VMEM((64, 128), jnp.float32),
            pltpu.SemaphoreType.DMA,
            pltpu.SemaphoreType.DMA,
            pltpu.SemaphoreType.DMA,
            pltpu.SemaphoreType.DMA,
            pltpu.SemaphoreType.DMA,
            pltpu.SemaphoreType.DMA,
            pltpu.SemaphoreType.DMA,
        ],
    )
    def k2(xt_hbm, lin_hbm, out_hbm, idxall, pidx, par64,
           pb0, pb1, pb2, ob0, ob1, ob2,
           gg0, gg1, gg2, go0, go1, go2, gs):
        wid = lax.axis_index("s") * _NC + lax.axis_index("c")
        iota = lax.iota(jnp.int32, 16)

        def stage(u, _):
            t = wid * per_w + u
            pltpu.async_copy(
                xt_hbm.at[pl.ds((t // nti) * 8, 8),
                          pl.ds((t % nti) * 128, 128)],
                idxall.at[:, pl.ds(u * 128, 128)], gs)
            return 0

        lax.fori_loop(0, per_w, stage, 0)

        def drain_stage(u, _):
            pltpu.make_async_copy(
                xt_hbm.at[pl.ds(0, 8), pl.ds(0, 128)],
                idxall.at[:, pl.ds(0, 128)], gs).wait()
            return 0

        lax.fori_loop(0, per_w, drain_stage, 0)

        def prep(g, s):
            u = g // 8
            j8 = g % 8

            def pv(m, _):
                v = idxall[j8, pl.ds(u * 128 + 16 * m, 16)]
                pidx[s, pl.ds(16 * m, 16)] = lax.shift_right_logical(v, 1)
                par64[s, pl.ds(16 * m, 16)] = (v & 1) * 64
                return 0

            lax.fori_loop(0, 8, pv, 0)

        def fire_gather(s, pb, sem):
            pltpu.async_copy(lin_hbm.at[pidx.at[s]], pb, sem)

        def transpose(pb, s, ob):
            def grp(m, _):
                parv = par64[s, pl.ds(16 * m, 16)]
                ridx = iota + 16 * m

                def db(t, _):
                    for u in range(4):
                        d = 4 * t + u
                        ob[d, pl.ds(16 * m, 16)] = plsc.load_gather(
                            pb, [ridx, parv + d])
                    return 0

                lax.fori_loop(0, D // 4, db, 0)
                return 0

            lax.fori_loop(0, 8, grp, 0)

        def out_slice(g):
            t = wid * per_w + g // 8
            return out_hbm.at[(t // nti) * 8 + g % 8, :,
                              pl.ds((t % nti) * 128, 128)]

        def half(g, s, pb, ob, gg, go):
            @pl.when(g < n_t)
            def _():
                pltpu.make_async_copy(lin_hbm.at[pidx.at[s]], pb, gg).wait()

                @pl.when(g >= 3)
                def _():
                    pltpu.make_async_copy(ob, out_slice(g - 3), go).wait()

                transpose(pb, s, ob)
                pltpu.async_copy(ob, out_slice(g), go)

            @pl.when(g + 3 < n_t)
            def _():
                prep(g + 3, s)
                fire_gather(s, pb, gg)

        for s in range(3):
            prep(s, s)
        fire_gather(0, pb0, gg0)
        fire_gather(1, pb1, gg1)
        fire_gather(2, pb2, gg2)

        def body(h, _):
            g = 3 * h
            half(g, 0, pb0, ob0, gg0, go0)
            half(g + 1, 1, pb1, ob1, gg1, go1)
            half(g + 2, 2, pb2, ob2, gg2, go2)
            return 0

        lax.fori_loop(0, (n_t + 2) // 3, body, 0)

        pltpu.make_async_copy(ob0, out_slice(0), go0).wait()
        pltpu.make_async_copy(ob1, out_slice(0), go1).wait()
        pltpu.make_async_copy(ob2, out_slice(0), go2).wait()

    return k2


def kernel(x, table):
    NI, NJ = x.shape
    xt = jnp.transpose(x.astype(jnp.int32))
    tt = jnp.transpose(table)
    tail = jnp.reshape(table[_FULL_COLS * 128:] * SCALE, (32, 128))
    lin = _make_k1()(tt, tail)
    outt = _make_k2(NJ, NI)(xt, lin)
    return jnp.transpose(outt, (2, 0, 1))


# speedup vs baseline: 1.9521x; 1.6069x over previous
import functools
import math

import jax
import jax.numpy as jnp
import numpy as np
from jax import lax
from jax.experimental import pallas as pl
from jax.experimental.pallas import tpu as pltpu
from jax.experimental.pallas import tpu_sc as plsc

D = 64
V = 1000000
SCALE = np.float32(math.sqrt(D))

_NC = 2
_NS = 16
_NW = _NC * _NS

_FULL_COLS = V // 128
_NPAIR = V // 2
_G = 4
_NBLK = _FULL_COLS // _G


def _make_k1():
    mesh = plsc.VectorSubcoreMesh(core_axis_name="c", subcore_axis_name="s")

    @functools.partial(
        pl.kernel,
        mesh=mesh,
        compiler_params=pltpu.CompilerParams(needs_layout_passes=False),
        out_type=jax.ShapeDtypeStruct((_NPAIR, 128), jnp.float32),
        scratch_types=[
            pltpu.VMEM((64, 128 * _G), jnp.float32),
            pltpu.VMEM((64, 128 * _G), jnp.float32),
            pltpu.VMEM((64 * _G, 128), jnp.float32),
            pltpu.VMEM((32, 128), jnp.float32),
            pltpu.SemaphoreType.DMA,
            pltpu.SemaphoreType.DMA,
            pltpu.SemaphoreType.DMA,
        ],
    )
    def k1(tt_hbm, tail_hbm, lin_hbm, vb0, vb1, nb, tbuf, gi0, gi1, go):
        wid = lax.axis_index("s") * _NC + lax.axis_index("c")

        @pl.when(wid == 0)
        def _():
            pltpu.sync_copy(tail_hbm, tbuf)
            pltpu.sync_copy(tbuf, lin_hbm.at[pl.ds(_FULL_COLS * 64, 32)])

        iota = lax.iota(jnp.int32, 16)
        n_b = jnp.where(wid < _NBLK % _NW, _NBLK // _NW + 1, _NBLK // _NW)

        def fire_in(m, vb, sem):
            blk = wid + m * _NW
            pltpu.async_copy(tt_hbm.at[:, pl.ds(blk * 128 * _G, 128 * _G)],
                             vb, sem)

        def transpose(vb):
            def pair_body(qq, _):
                c0 = jnp.full((16,), 2 * qq, jnp.int32)
                c1 = c0 + 1
                vals = []
                for k in range(8):
                    ridx = iota + (16 * (k % 4))
                    cidx = c1 if k >= 4 else c0
                    vals.append(plsc.load_gather(vb, [ridx, cidx]) * SCALE)
                for k in range(8):
                    nb[qq, pl.ds(16 * k, 16)] = vals[k]
                return 0

            lax.fori_loop(0, 64 * _G, pair_body, 0)

        def half(m, vb, gi):
            @pl.when(m < n_b)
            def _():
                blk = wid + m * _NW
                pltpu.make_async_copy(
                    tt_hbm.at[:, pl.ds(blk * 128 * _G, 128 * _G)], vb,
                    gi).wait()

                @pl.when(m >= 1)
                def _():
                    pltpu.make_async_copy(
                        nb, lin_hbm.at[pl.ds(blk * 64 * _G, 64 * _G)],
                        go).wait()

                transpose(vb)
                pltpu.async_copy(
                    nb, lin_hbm.at[pl.ds(blk * 64 * _G, 64 * _G)], go)

            @pl.when(m + 2 < n_b)
            def _():
                fire_in(m + 2, vb, gi)

        fire_in(0, vb0, gi0)
        fire_in(1, vb1, gi1)

        def body(h, _):
            half(2 * h, vb0, gi0)
            half(2 * h + 1, vb1, gi1)
            return 0

        lax.fori_loop(0, (_NBLK // _NW + 2) // 2, body, 0)
        pltpu.make_async_copy(nb, lin_hbm.at[pl.ds(0, 64 * _G)], go).wait()

    return k1


def _make_k2(NJ: int, NI: int):
    njb = NJ // 8
    nti = NI // 128
    units = njb * nti
    per_w = units // _NW
    n_t = per_w * 8
    mesh = plsc.VectorSubcoreMesh(core_axis_name="c", subcore_axis_name="s")

    @functools.partial(
        pl.kernel,
        mesh=mesh,
        compiler_params=pltpu.CompilerParams(needs_layout_passes=False),
        out_type=jax.ShapeDtypeStruct((NJ, D, NI), jnp.float32),
        scratch_types=[
            pltpu.VMEM((8, per_w * 128), jnp.int32),
            pltpu.VMEM((3, 128), jnp.int32),
            pltpu.VMEM((3, 128), jnp.int32),
            pltpu.VMEM((128, 128), jnp.float32),
            pltpu.VMEM((128, 128), jnp.float32),
            pltpu.VMEM((128, 128), jnp.float32),
            pltpu.VMEM((64, 128), jnp.float32),
            pltpu.VMEM((64, 128), jnp.float32),
            pltpu.VMEM((64, 128), jnp.float32),
            pltpu.SemaphoreType.DMA,
            pltpu.SemaphoreType.DMA,
            pltpu.SemaphoreType.DMA,
            pltpu.SemaphoreType.DMA,
            pltpu.SemaphoreType.DMA,
            pltpu.SemaphoreType.DMA,
            pltpu.SemaphoreType.DMA,
        ],
    )
    def k2(xt_hbm, lin_hbm, out_hbm, idxall, pidx, par64,
           pb0, pb1, pb2, ob0, ob1, ob2,
           gg0, gg1, gg2, go0, go1, go2, gs):
        wid = lax.axis_index("s") * _NC + lax.axis_index("c")
        iota = lax.iota(jnp.int32, 16)

        def stage(u, _):
            t = wid * per_w + u
            pltpu.async_copy(
                xt_hbm.at[pl.ds((t // nti) * 8, 8),
                          pl.ds((t % nti) * 128, 128)],
                idxall.at[:, pl.ds(u * 128, 128)], gs)
            return 0

        lax.fori_loop(0, per_w, stage, 0)

        def drain_stage(u, _):
            pltpu.make_async_copy(
                xt_hbm.at[pl.ds(0, 8), pl.ds(0, 128)],
                idxall.at[:, pl.ds(0, 128)], gs).wait()
            return 0

        lax.fori_loop(0, per_w, drain_stage, 0)

        def prep(g, s):
            u = g // 8
            j8 = g % 8

            def pv(m, _):
                v = idxall[j8, pl.ds(u * 128 + 16 * m, 16)]
                pidx[s, pl.ds(16 * m, 16)] = lax.shift_right_logical(v, 1)
                par64[s, pl.ds(16 * m, 16)] = (v & 1) * 64
                return 0

            lax.fori_loop(0, 8, pv, 0)

        def fire_gather(s, pb, sem):
            pltpu.async_copy(lin_hbm.at[pidx.at[s]], pb, sem)

        def transpose(pb, s, ob):
            def grp(m, _):
                parv = par64[s, pl.ds(16 * m, 16)]
                ridx = iota + 16 * m

                def db(t, _):
                    d0 = 8 * t
                    vals = [plsc.load_gather(pb, [ridx, parv + d0 + u])
                            for u in range(8)]
                    for u in range(8):
                        ob[d0 + u, pl.ds(16 * m, 16)] = vals[u]
                    return 0

                lax.fori_loop(0, D // 8, db, 0)
                return 0

            lax.fori_loop(0, 8, grp, 0)

        def out_slice(g):
            t = wid * per_w + g // 8
            return out_hbm.at[(t // nti) * 8 + g % 8, :,
                              pl.ds((t % nti) * 128, 128)]

        def half(g, s, pb, ob, gg, go):
            @pl.when(g < n_t)
            def _():
                pltpu.make_async_copy(lin_hbm.at[pidx.at[s]], pb, gg).wait()

                @pl.when(g >= 3)
                def _():
                    pltpu.make_async_copy(ob, out_slice(g - 3), go).wait()

                transpose(pb, s, ob)
                pltpu.async_copy(ob, out_slice(g), go)

            @pl.when(g + 3 < n_t)
            def _():
                prep(g + 3, s)
                fire_gather(s, pb, gg)

        for s in range(3):
            prep(s, s)
        fire_gather(0, pb0, gg0)
        fire_gather(1, pb1, gg1)
        fire_gather(2, pb2, gg2)

        def body(h, _):
            g = 3 * h
            half(g, 0, pb0, ob0, gg0, go0)
            half(g + 1, 1, pb1, ob1, gg1, go1)
            half(g + 2, 2, pb2, ob2, gg2, go2)
            return 0

        lax.fori_loop(0, (n_t + 2) // 3, body, 0)

        pltpu.make_async_copy(ob0, out_slice(0), go0).wait()
        pltpu.make_async_copy(ob1, out_slice(0), go1).wait()
        pltpu.make_async_copy(ob2, out_slice(0), go2).wait()

    return k2


def kernel(x, table):
    NI, NJ = x.shape
    xt = jnp.transpose(x.astype(jnp.int32))
    tt = jnp.transpose(table)
    tail = jnp.reshape(table[_FULL_COLS * 128:] * SCALE, (32, 128))
    lin = _make_k1()(tt, tail)
    outt = _make_k2(NJ, NI)(xt, lin)
    return jnp.transpose(outt, (2, 0, 1))


# speedup vs baseline: 1.9993x; 1.0242x over previous
import functools
import math

import jax
import jax.numpy as jnp
import numpy as np
from jax import lax
from jax.experimental import pallas as pl
from jax.experimental.pallas import tpu as pltpu
from jax.experimental.pallas import tpu_sc as plsc

D = 64
V = 1000000
SCALE = np.float32(math.sqrt(D))

_NC = 2
_NS = 16
_NW = _NC * _NS

_FULL_COLS = V // 128
_NPAIR = V // 2
_G = 4
_NBLK = _FULL_COLS // _G


def _make_k1():
    mesh = plsc.VectorSubcoreMesh(core_axis_name="c", subcore_axis_name="s")

    @functools.partial(
        pl.kernel,
        mesh=mesh,
        compiler_params=pltpu.CompilerParams(needs_layout_passes=False),
        out_type=jax.ShapeDtypeStruct((_NPAIR, 128), jnp.float32),
        scratch_types=[
            pltpu.VMEM((64, 128), jnp.float32),
            pltpu.VMEM((64, 128), jnp.float32),
            pltpu.VMEM((64, 128), jnp.float32),
            pltpu.VMEM((64, 128), jnp.float32),
            pltpu.VMEM((64, 128), jnp.float32),
            pltpu.VMEM((64, 128), jnp.float32),
            pltpu.VMEM((32, 128), jnp.float32),
            pltpu.SemaphoreType.DMA,
            pltpu.SemaphoreType.DMA,
            pltpu.SemaphoreType.DMA,
            pltpu.SemaphoreType.DMA,
            pltpu.SemaphoreType.DMA,
            pltpu.SemaphoreType.DMA,
        ],
    )
    def k1(tt_hbm, tail_hbm, lin_hbm, vb0, vb1, vb2, vb3, nb0, nb1, tbuf,
           gi0, gi1, gi2, gi3, go0, go1):
        wid = lax.axis_index("s") * _NC + lax.axis_index("c")

        @pl.when(wid == 0)
        def _():
            pltpu.sync_copy(tail_hbm, tbuf)
            pltpu.sync_copy(tbuf, lin_hbm.at[pl.ds(_FULL_COLS * 64, 32)])

        iota = lax.iota(jnp.int32, 16)
        n_c = jnp.where(wid < _FULL_COLS % _NW,
                        _FULL_COLS // _NW + 1, _FULL_COLS // _NW)

        def fire_in(m, vb, sem):
            tc = wid + m * _NW
            pltpu.async_copy(tt_hbm.at[:, pl.ds(tc * 128, 128)], vb, sem)

        def transpose(vb, nb):
            def pair_body(q, _):
                c0 = jnp.full((16,), 2 * q, jnp.int32)
                c1 = c0 + 1
                vals = []
                for k in range(8):
                    ridx = iota + (16 * (k % 4))
                    cidx = c1 if k >= 4 else c0
                    vals.append(plsc.load_gather(vb, [ridx, cidx]) * SCALE)
                for k in range(8):
                    nb[q, pl.ds(16 * k, 16)] = vals[k]
                return 0

            lax.fori_loop(0, 64, pair_body, 0)

        def half(m, vb, gi, nb, go):
            @pl.when(m < n_c)
            def _():
                tc = wid + m * _NW
                pltpu.make_async_copy(
                    tt_hbm.at[:, pl.ds(tc * 128, 128)], vb, gi).wait()

                @pl.when(m >= 2)
                def _():
                    pltpu.make_async_copy(
                        nb, lin_hbm.at[pl.ds(tc * 64, 64)], go).wait()

                transpose(vb, nb)
                pltpu.async_copy(nb, lin_hbm.at[pl.ds(tc * 64, 64)], go)

            @pl.when(m + 4 < n_c)
            def _():
                fire_in(m + 4, vb, gi)

        fire_in(0, vb0, gi0)
        fire_in(1, vb1, gi1)
        fire_in(2, vb2, gi2)
        fire_in(3, vb3, gi3)

        def body(h, _):
            m = 4 * h
            half(m, vb0, gi0, nb0, go0)
            half(m + 1, vb1, gi1, nb1, go1)
            half(m + 2, vb2, gi2, nb0, go0)
            half(m + 3, vb3, gi3, nb1, go1)
            return 0

        lax.fori_loop(0, (_FULL_COLS // _NW + 4) // 4, body, 0)
        pltpu.make_async_copy(nb0, lin_hbm.at[pl.ds(0, 64)], go0).wait()
        pltpu.make_async_copy(nb1, lin_hbm.at[pl.ds(0, 64)], go1).wait()

    return k1


def _make_k2(NJ: int, NI: int):
    njb = NJ // 8
    nti = NI // 128
    units = njb * nti
    per_w = units // _NW
    n_t = per_w * 8
    mesh = plsc.VectorSubcoreMesh(core_axis_name="c", subcore_axis_name="s")

    @functools.partial(
        pl.kernel,
        mesh=mesh,
        compiler_params=pltpu.CompilerParams(needs_layout_passes=False),
        out_type=jax.ShapeDtypeStruct((NJ, D, NI), jnp.float32),
        scratch_types=[
            pltpu.VMEM((8, per_w * 128), jnp.int32),
            pltpu.VMEM((4, 128), jnp.int32),
            pltpu.VMEM((4, 128), jnp.int32),
            pltpu.VMEM((128, 128), jnp.float32),
            pltpu.VMEM((128, 128), jnp.float32),
            pltpu.VMEM((128, 128), jnp.float32),
            pltpu.VMEM((128, 128), jnp.float32),
            pltpu.VMEM((64, 128), jnp.float32),
            pltpu.VMEM((64, 128), jnp.float32),
            pltpu.SemaphoreType.DMA,
            pltpu.SemaphoreType.DMA,
            pltpu.SemaphoreType.DMA,
            pltpu.SemaphoreType.DMA,
            pltpu.SemaphoreType.DMA,
            pltpu.SemaphoreType.DMA,
            pltpu.SemaphoreType.DMA,
        ],
    )
    def k2(xt_hbm, lin_hbm, out_hbm, idxall, pidx, par64,
           pb0, pb1, pb2, pb3, ob0, ob1,
           gg0, gg1, gg2, gg3, go0, go1, gs):
        wid = lax.axis_index("s") * _NC + lax.axis_index("c")
        iota = lax.iota(jnp.int32, 16)

        def stage(u, _):
            t = wid * per_w + u
            pltpu.async_copy(
                xt_hbm.at[pl.ds((t // nti) * 8, 8),
                          pl.ds((t % nti) * 128, 128)],
                idxall.at[:, pl.ds(u * 128, 128)], gs)
            return 0

        lax.fori_loop(0, per_w, stage, 0)

        def drain_stage(u, _):
            pltpu.make_async_copy(
                xt_hbm.at[pl.ds(0, 8), pl.ds(0, 128)],
                idxall.at[:, pl.ds(0, 128)], gs).wait()
            return 0

        lax.fori_loop(0, per_w, drain_stage, 0)

        def prep(g, s):
            u = g // 8
            j8 = g % 8

            def pv(m, _):
                v = idxall[j8, pl.ds(u * 128 + 16 * m, 16)]
                pidx[s, pl.ds(16 * m, 16)] = lax.shift_right_logical(v, 1)
                par64[s, pl.ds(16 * m, 16)] = (v & 1) * 64
                return 0

            lax.fori_loop(0, 8, pv, 0)

        def fire_gather(s, pb, sem):
            pltpu.async_copy(lin_hbm.at[pidx.at[s]], pb, sem)

        def transpose(pb, s, ob):
            def grp(m, _):
                parv = par64[s, pl.ds(16 * m, 16)]
                ridx = iota + 16 * m

                def db(t, _):
                    d0 = 8 * t
                    vals = [plsc.load_gather(pb, [ridx, parv + d0 + u])
                            for u in range(8)]
                    for u in range(8):
                        ob[d0 + u, pl.ds(16 * m, 16)] = vals[u]
                    return 0

                lax.fori_loop(0, D // 8, db, 0)
                return 0

            lax.fori_loop(0, 8, grp, 0)

        def out_slice(g):
            t = wid * per_w + g // 8
            return out_hbm.at[(t // nti) * 8 + g % 8, :,
                              pl.ds((t % nti) * 128, 128)]

        def half(g, s, pb, ob, gg, go):
            @pl.when(g < n_t)
            def _():
                pltpu.make_async_copy(lin_hbm.at[pidx.at[s]], pb, gg).wait()

                @pl.when(g >= 2)
                def _():
                    pltpu.make_async_copy(ob, out_slice(g - 2), go).wait()

                transpose(pb, s, ob)
                pltpu.async_copy(ob, out_slice(g), go)

            @pl.when(g + 4 < n_t)
            def _():
                prep(g + 4, s)
                fire_gather(s, pb, gg)

        for s in range(4):
            prep(s, s)
        fire_gather(0, pb0, gg0)
        fire_gather(1, pb1, gg1)
        fire_gather(2, pb2, gg2)
        fire_gather(3, pb3, gg3)

        def body(h, _):
            g = 4 * h
            half(g, 0, pb0, ob0, gg0, go0)
            half(g + 1, 1, pb1, ob1, gg1, go1)
            half(g + 2, 2, pb2, ob0, gg2, go0)
            half(g + 3, 3, pb3, ob1, gg3, go1)
            return 0

        lax.fori_loop(0, n_t // 4, body, 0)

        pltpu.make_async_copy(ob0, out_slice(0), go0).wait()
        pltpu.make_async_copy(ob1, out_slice(0), go1).wait()

    return k2


def kernel(x, table):
    NI, NJ = x.shape
    xt = jnp.transpose(x.astype(jnp.int32))
    tt = jnp.transpose(table)
    tail = jnp.reshape(table[_FULL_COLS * 128:] * SCALE, (32, 128))
    lin = _make_k1()(tt, tail)
    outt = _make_k2(NJ, NI)(xt, lin)
    return jnp.transpose(outt, (2, 0, 1))


# speedup vs baseline: 2.9713x; 1.4862x over previous
import functools
import math

import jax
import jax.numpy as jnp
import numpy as np
from jax import lax
from jax.experimental import pallas as pl
from jax.experimental.pallas import tpu as pltpu
from jax.experimental.pallas import tpu_sc as plsc

D_MODEL = 64
SCALE = np.float32(math.sqrt(D_MODEL))

_NC = 2
_NS = 16
_NW = _NC * _NS
_CHUNK = 128


def _make_gather(B: int, D: int):
    assert B % (_NW * _CHUNK) == 0
    n_chunks = B // (_NW * _CHUNK)
    mesh = plsc.VectorSubcoreMesh(core_axis_name="c", subcore_axis_name="s")

    @functools.partial(
        pl.kernel,
        mesh=mesh,
        compiler_params=pltpu.CompilerParams(use_tc_tiling_on_sc=False),
        out_type=jax.ShapeDtypeStruct((B, D), jnp.float32),
        scratch_types=[
            pltpu.VMEM((n_chunks, _CHUNK), jnp.int32),
            pltpu.VMEM((_CHUNK, D), jnp.float32),
            pltpu.VMEM((_CHUNK, D), jnp.float32),
            pltpu.VMEM((_CHUNK, D), jnp.float32),
            pltpu.VMEM((_CHUNK, D), jnp.float32),
            pltpu.VMEM((_CHUNK, D), jnp.float32),
            pltpu.VMEM((_CHUNK, D), jnp.float32),
            pltpu.SemaphoreType.DMA,
            pltpu.SemaphoreType.DMA,
            pltpu.SemaphoreType.DMA,
            pltpu.SemaphoreType.DMA,
            pltpu.SemaphoreType.DMA,
            pltpu.SemaphoreType.DMA,
        ],
    )
    def gather_scale(idx_hbm, table_hbm, out_hbm, idx_v,
                     pb0, pb1, pb2, pb3, ob0, ob1,
                     gg0, gg1, gg2, gg3, go0, go1):
        wid = lax.axis_index("s") * _NC + lax.axis_index("c")
        row0 = wid * n_chunks

        pltpu.sync_copy(idx_hbm.at[pl.ds(row0, n_chunks)], idx_v)

        def fire_gather(g, pb, sem):
            pltpu.async_copy(table_hbm.at[idx_v.at[g]], pb, sem)

        def scale(pb, ob):
            def row(r, _):
                vals = [pb[r, pl.ds(16 * c, 16)] * SCALE
                        for c in range(D // 16)]
                for c in range(D // 16):
                    ob[r, pl.ds(16 * c, 16)] = vals[c]
                return 0

            lax.fori_loop(0, _CHUNK, row, 0)

        def out_slice(g):
            return out_hbm.at[pl.ds((row0 + g) * _CHUNK, _CHUNK)]

        def half(g, pb, ob, gg, go):
            @pl.when(g < n_chunks)
            def _():
                pltpu.make_async_copy(
                    table_hbm.at[idx_v.at[g]], pb, gg).wait()

                @pl.when(g >= 2)
                def _():
                    pltpu.make_async_copy(ob, out_slice(g - 2), go).wait()

                scale(pb, ob)
                pltpu.async_copy(ob, out_slice(g), go)

            @pl.when(g + 4 < n_chunks)
            def _():
                fire_gather(g + 4, pb, gg)

        fire_gather(0, pb0, gg0)
        fire_gather(1, pb1, gg1)
        fire_gather(2, pb2, gg2)
        fire_gather(3, pb3, gg3)

        def body(h, _):
            g = 4 * h
            half(g, pb0, ob0, gg0, go0)
            half(g + 1, pb1, ob1, gg1, go1)
            half(g + 2, pb2, ob0, gg2, go0)
            half(g + 3, pb3, ob1, gg3, go1)
            return 0

        lax.fori_loop(0, n_chunks // 4, body, 0)
        pltpu.make_async_copy(ob0, out_slice(0), go0).wait()
        pltpu.make_async_copy(ob1, out_slice(0), go1).wait()

    return gather_scale


def kernel(x, table):
    orig_shape = x.shape
    B = x.size
    idx = jnp.reshape(x.astype(jnp.int32), (B // _CHUNK, _CHUNK))
    out = _make_gather(B, D_MODEL)(idx, table)
    return jnp.reshape(out, orig_shape + (D_MODEL,))
